# baseline (speedup 1.0000x reference)
import functools

import jax
import jax.numpy as jnp
from jax import lax
from jax.experimental import pallas as pl

N = 10000
E = 320000
D = 128
BLK = 1000
GRID = N // BLK


def _dense_pre_body(x_ref, wp_ref, w0_ref, w1_ref, al0_ref, ar0_ref,
                    al1_ref, ar1_ref, z0_ref, z1_ref, elr_ref, gmax_ref,
                    mx_ref):
    i = pl.program_id(0)
    h = x_ref[...] @ wp_ref[...]
    z0 = h @ w0_ref[...]
    z1 = h @ w1_ref[...]
    z0_ref[...] = z0
    z1_ref[...] = z1
    el0 = z0 @ al0_ref[...].T
    er0 = z0 @ ar0_ref[...].T
    el1 = z1 @ al1_ref[...].T
    er1 = z1 @ ar1_ref[...].T
    cols = jnp.concatenate([el0, er0, el1, er1], axis=1)
    elr_ref[...] = cols
    m = jnp.max(cols, axis=0)[None, :]
    prev = jnp.where(i == 0, jnp.full((1, 4), -jnp.inf, jnp.float32),
                     mx_ref[...])
    mx = jnp.maximum(m, prev)
    mx_ref[...] = mx
    g0 = mx[0, 0] + mx[0, 1]
    g1 = mx[0, 2] + mx[0, 3]
    g0 = jnp.maximum(g0, 0.2 * g0)
    g1 = jnp.maximum(g1, 0.2 * g1)
    gmax_ref[...] = jnp.stack([jnp.full((16,), g0, jnp.float32),
                               jnp.full((16,), g1, jnp.float32)])


def _dense_pre(x, W_proj, W_fc0, W_fc1, al0, ar0, al1, ar1):
    from jax.experimental.pallas import tpu as pltpu
    return pl.pallas_call(
        _dense_pre_body,
        grid=(GRID,),
        in_specs=[
            pl.BlockSpec((BLK, D), lambda i: (i, 0)),
            pl.BlockSpec((D, D), lambda i: (0, 0)),
            pl.BlockSpec((D, D), lambda i: (0, 0)),
            pl.BlockSpec((D, D), lambda i: (0, 0)),
            pl.BlockSpec((1, D), lambda i: (0, 0)),
            pl.BlockSpec((1, D), lambda i: (0, 0)),
            pl.BlockSpec((1, D), lambda i: (0, 0)),
            pl.BlockSpec((1, D), lambda i: (0, 0)),
        ],
        out_specs=[
            pl.BlockSpec((BLK, D), lambda i: (i, 0)),
            pl.BlockSpec((BLK, D), lambda i: (i, 0)),
            pl.BlockSpec((BLK, 4), lambda i: (i, 0)),
            pl.BlockSpec((2, 16), lambda i: (0, 0)),
        ],
        out_shape=[
            jax.ShapeDtypeStruct((N, D), jnp.float32),
            jax.ShapeDtypeStruct((N, D), jnp.float32),
            jax.ShapeDtypeStruct((N, 4), jnp.float32),
            jax.ShapeDtypeStruct((2, 16), jnp.float32),
        ],
        scratch_shapes=[pltpu.VMEM((1, 4), jnp.float32)],
    )(x, W_proj, W_fc0, W_fc1, al0, ar0, al1, ar1)


def kernel(x, edge_index_mp0, edge_index_mp1, W_proj, W_fc0, attn_l0,
           attn_r0, W_fc1, attn_l1, attn_r1, W_s1, b_s1, W_s2):
    z0, z1, elr, gmax = _dense_pre(x, W_proj, W_fc0, W_fc1,
                                   attn_l0, attn_r0, attn_l1, attn_r1)

    def sparse_mid(z, src, dst, mp):
        e = elr[src, 2 * mp] + elr[dst, 2 * mp + 1]
        e = jnp.maximum(e, 0.2 * e)
        a = jnp.exp(e - gmax[mp, 0])
        U = jax.ops.segment_sum(a[:, None] * z[src], dst, num_segments=N)
        s = jax.ops.segment_sum(a, dst, num_segments=N)
        return U, s

    U0, s0 = sparse_mid(z0, edge_index_mp0[0], edge_index_mp0[1], 0)
    U1, s1 = sparse_mid(z1, edge_index_mp1[0], edge_index_mp1[1], 1)

    zo0 = jax.nn.elu(U0 / (s0[:, None] + 1e-9))
    zo1 = jax.nn.elu(U1 / (s1[:, None] + 1e-9))
    w0 = jnp.mean(jnp.tanh(zo0 @ W_s1 + b_s1) @ W_s2, axis=0)
    w1 = jnp.mean(jnp.tanh(zo1 @ W_s1 + b_s1) @ W_s2, axis=0)
    beta = jax.nn.softmax(jnp.stack([w0, w1], axis=0), axis=0)
    return beta[0] * zo0 + beta[1] * zo1


# speedup vs baseline: 33.3547x; 33.3547x over previous
import functools

import jax
import jax.numpy as jnp
from jax import lax
from jax.experimental import pallas as pl
from jax.experimental.pallas import tpu as pltpu
from jax.experimental.pallas import tpu_sc as plsc

N = 10000
E = 320000
D = 128
BLK = 1000
GRID = N // BLK

NTILE = 16
CH = 128
NCK = E // CH
RPT = N // NTILE


def _dense_pre_body(x_ref, wp_ref, w0_ref, w1_ref, al0_ref, ar0_ref,
                    al1_ref, ar1_ref, zs_ref, elr_ref, gmax_ref, mx_ref):
    i = pl.program_id(0)
    h = x_ref[...] @ wp_ref[...]
    z0 = h @ w0_ref[...]
    z1 = h @ w1_ref[...]
    zs_ref[0] = z0
    zs_ref[1] = z1
    el0 = z0 @ al0_ref[...].T
    er0 = z0 @ ar0_ref[...].T
    el1 = z1 @ al1_ref[...].T
    er1 = z1 @ ar1_ref[...].T
    cols = jnp.concatenate(
        [el0, er0, el1, er1, jnp.zeros((BLK, 12), jnp.float32)], axis=1)
    elr_ref[...] = cols
    m = jnp.max(cols, axis=0)[None, :]
    prev = jnp.where(i == 0, jnp.full((1, 16), -jnp.inf, jnp.float32),
                     mx_ref[...])
    mx = jnp.maximum(m, prev)
    mx_ref[...] = mx
    g0 = mx[0, 0] + mx[0, 1]
    g1 = mx[0, 2] + mx[0, 3]
    g0 = jnp.maximum(g0, 0.2 * g0)
    g1 = jnp.maximum(g1, 0.2 * g1)
    gmax_ref[...] = jnp.stack([jnp.full((16,), g0, jnp.float32),
                               jnp.full((16,), g1, jnp.float32)])


def _dense_pre(x, W_proj, W_fc0, W_fc1, al0, ar0, al1, ar1):
    return pl.pallas_call(
        _dense_pre_body,
        grid=(GRID,),
        in_specs=[
            pl.BlockSpec((BLK, D), lambda i: (i, 0)),
            pl.BlockSpec((D, D), lambda i: (0, 0)),
            pl.BlockSpec((D, D), lambda i: (0, 0)),
            pl.BlockSpec((D, D), lambda i: (0, 0)),
            pl.BlockSpec((1, D), lambda i: (0, 0)),
            pl.BlockSpec((1, D), lambda i: (0, 0)),
            pl.BlockSpec((1, D), lambda i: (0, 0)),
            pl.BlockSpec((1, D), lambda i: (0, 0)),
        ],
        out_specs=[
            pl.BlockSpec((2, BLK, D), lambda i: (0, i, 0)),
            pl.BlockSpec((BLK, 16), lambda i: (i, 0)),
            pl.BlockSpec((2, 16), lambda i: (0, 0)),
        ],
        out_shape=[
            jax.ShapeDtypeStruct((2, N, D), jnp.float32),
            jax.ShapeDtypeStruct((N, 16), jnp.float32),
            jax.ShapeDtypeStruct((2, 16), jnp.float32),
        ],
        scratch_shapes=[pltpu.VMEM((1, 16), jnp.float32)],
    )(x, W_proj, W_fc0, W_fc1, al0, ar0, al1, ar1)


def _sc_body(zs_h, ed_h, elr_h, gmax_h, zu_h, zs0_h,
             u_out, s_out,
             gmax_v, srcc_v, dstc_v, a_v, elbuf, erbuf, zbuf, srow,
             U_sh, S_sh, semz, seml, semr):
    c = lax.axis_index("c")
    s = lax.axis_index("s")
    r0 = s * RPT

    pltpu.sync_copy(gmax_h.at[c], gmax_v)
    pltpu.sync_copy(zu_h.at[pl.ds(r0, RPT)], U_sh.at[pl.ds(r0, RPT)])
    pltpu.sync_copy(zs0_h.at[pl.ds(r0, RPT)], S_sh.at[pl.ds(r0, RPT)])

    zf16 = jnp.zeros((16,), jnp.float32)

    def _zr(i, _):
        srow[i, :] = zf16
        return 0
    lax.fori_loop(0, CH, _zr, 0)

    gm = gmax_v[...]
    iota16 = lax.iota(jnp.int32, 16)
    zi16 = jnp.zeros((16,), jnp.int32)
    cel = jnp.broadcast_to(2 * c, (16,)).astype(jnp.int32)
    cer = cel + 1

    plsc.subcore_barrier()

    nck = jnp.where(s < NCK % NTILE, NCK // NTILE + 1, NCK // NTILE)

    def chunk(t, _):
        base = (s + t * NTILE) * CH
        pltpu.sync_copy(ed_h.at[c, 0, pl.ds(base, CH)], srcc_v)
        pltpu.sync_copy(ed_h.at[c, 1, pl.ds(base, CH)], dstc_v)
        gz = pltpu.async_copy(zs_h.at[c].at[srcc_v], zbuf, semz)
        gl = pltpu.async_copy(elr_h.at[srcc_v], elbuf, seml)
        gr = pltpu.async_copy(elr_h.at[dstc_v], erbuf, semr)
        gl.wait()
        gr.wait()
        for j in range(CH // 16):
            ridx = iota16 + j * 16
            el = plsc.load_gather(elbuf, [ridx, cel])
            er = plsc.load_gather(erbuf, [ridx, cer])
            xx = el + er
            e = jnp.maximum(xx, 0.2 * xx)
            a = jnp.exp(e - gm)
            a_v[pl.ds(j * 16, 16)] = a
            plsc.store_scatter(srow, [ridx, zi16], a)
        gz.wait()

        def scale_row(i, _):
            asp = plsc.load_gather(a_v, [jnp.broadcast_to(i, (16,))])
            for kk in range(D // 16):
                zbuf[i, pl.ds(kk * 16, 16)] = (
                    zbuf[i, pl.ds(kk * 16, 16)] * asp)
            return 0
        lax.fori_loop(0, CH, scale_row, 0)

        pltpu.sync_copy(zbuf, U_sh.at[dstc_v], add=True)
        pltpu.sync_copy(srow, S_sh.at[dstc_v], add=True)
        return 0

    lax.fori_loop(0, nck, chunk, 0)

    plsc.subcore_barrier()
    pltpu.sync_copy(U_sh.at[pl.ds(r0, RPT)], u_out.at[c, pl.ds(r0, RPT)])
    pltpu.sync_copy(S_sh.at[pl.ds(r0, RPT)], s_out.at[c, pl.ds(r0, RPT)])


def _sc_gat(zs, ed, elr, gmax, zu, zs0):
    mesh = plsc.VectorSubcoreMesh(core_axis_name="c", subcore_axis_name="s")
    f = pl.kernel(
        _sc_body,
        out_type=(jax.ShapeDtypeStruct((2, N, D), jnp.float32),
                  jax.ShapeDtypeStruct((2, N, 16), jnp.float32)),
        mesh=mesh,
        compiler_params=pltpu.CompilerParams(use_tc_tiling_on_sc=False,
                                             needs_layout_passes=False),
        scratch_types=[
            pltpu.VMEM((16,), jnp.float32),
            pltpu.VMEM((CH,), jnp.int32),
            pltpu.VMEM((CH,), jnp.int32),
            pltpu.VMEM((CH,), jnp.float32),
            pltpu.VMEM((CH, 16), jnp.float32),
            pltpu.VMEM((CH, 16), jnp.float32),
            pltpu.VMEM((CH, D), jnp.float32),
            pltpu.VMEM((CH, 16), jnp.float32),
            pltpu.VMEM_SHARED((N, D), jnp.float32),
            pltpu.VMEM_SHARED((N, 16), jnp.float32),
            pltpu.SemaphoreType.DMA,
            pltpu.SemaphoreType.DMA,
            pltpu.SemaphoreType.DMA,
        ],
    )
    return f(zs, ed, elr, gmax, zu, zs0)


def _z_from(u_ref, s_ref, mp):
    z = u_ref[mp] / (s_ref[mp, :, 0][:, None] + 1e-9)
    return jnp.where(z > 0, z, jnp.exp(jnp.minimum(z, 0.0)) - 1.0)


def _sem_w_body(u_ref, s_ref, ws1_ref, b_ref, ws2t_ref, w_ref, acc_ref):
    i = pl.program_id(0)
    z0 = _z_from(u_ref, s_ref, 0)
    z1 = _z_from(u_ref, s_ref, 1)
    t0 = jnp.sum(jnp.tanh(z0 @ ws1_ref[...] + b_ref[...]) * ws2t_ref[...])
    t1 = jnp.sum(jnp.tanh(z1 @ ws1_ref[...] + b_ref[...]) * ws2t_ref[...])
    cur = jnp.stack([t0, t1])[None, :]
    prev = jnp.where(i == 0, jnp.zeros((1, 2), jnp.float32), acc_ref[...])
    acc = prev + cur
    acc_ref[...] = acc
    w_ref[...] = acc / N


def _combine_body(u_ref, s_ref, w_ref, out_ref):
    z0 = _z_from(u_ref, s_ref, 0)
    z1 = _z_from(u_ref, s_ref, 1)
    w0 = w_ref[0, 0]
    w1 = w_ref[0, 1]
    m = jnp.maximum(w0, w1)
    b0 = jnp.exp(w0 - m)
    b1 = jnp.exp(w1 - m)
    t = b0 + b1
    out_ref[...] = (b0 / t) * z0 + (b1 / t) * z1


def _combine(u, s_, W_s1, b_s1, W_s2):
    b2d = b_s1.reshape(1, D)
    ws2t = W_s2.reshape(1, D)
    u_spec = pl.BlockSpec((2, BLK, D), lambda i: (0, i, 0))
    s_spec = pl.BlockSpec((2, BLK, 16), lambda i: (0, i, 0))
    w = pl.pallas_call(
        _sem_w_body,
        grid=(GRID,),
        in_specs=[
            u_spec, s_spec,
            pl.BlockSpec((D, D), lambda i: (0, 0)),
            pl.BlockSpec((1, D), lambda i: (0, 0)),
            pl.BlockSpec((1, D), lambda i: (0, 0)),
        ],
        out_specs=pl.BlockSpec((1, 2), lambda i: (0, 0)),
        out_shape=jax.ShapeDtypeStruct((1, 2), jnp.float32),
        scratch_shapes=[pltpu.VMEM((1, 2), jnp.float32)],
    )(u, s_, W_s1, b2d, ws2t)
    return pl.pallas_call(
        _combine_body,
        grid=(GRID,),
        in_specs=[u_spec, s_spec, pl.BlockSpec((1, 2), lambda i: (0, 0))],
        out_specs=pl.BlockSpec((BLK, D), lambda i: (i, 0)),
        out_shape=jax.ShapeDtypeStruct((N, D), jnp.float32),
    )(u, s_, w)


def kernel(x, edge_index_mp0, edge_index_mp1, W_proj, W_fc0, attn_l0,
           attn_r0, W_fc1, attn_l1, attn_r1, W_s1, b_s1, W_s2):
    zs, elr, gmax = _dense_pre(x, W_proj, W_fc0, W_fc1,
                               attn_l0, attn_r0, attn_l1, attn_r1)
    zu = jnp.zeros((N, D), jnp.float32)
    zs0 = jnp.zeros((N, 16), jnp.float32)
    ed = jnp.stack([edge_index_mp0, edge_index_mp1])
    u, s_ = _sc_gat(zs, ed, elr, gmax, zu, zs0)
    return _combine(u, s_, W_s1, b_s1, W_s2)


# speedup vs baseline: 53.0968x; 1.5919x over previous
import functools

import jax
import jax.numpy as jnp
from jax import lax
from jax.experimental import pallas as pl
from jax.experimental.pallas import tpu as pltpu
from jax.experimental.pallas import tpu_sc as plsc

N = 10000
E = 320000
D = 128
BLK = 1000
GRID = N // BLK

NTILE = 16
CH = 128
NCK = E // CH
TPC = -(-NCK // NTILE)
SW = 8
RPT = N // NTILE


def _dense_pre_body(x_ref, wp_ref, w0_ref, w1_ref, al0_ref, ar0_ref,
                    al1_ref, ar1_ref, zs_ref, elr_ref, gmax_ref, mx_ref):
    i = pl.program_id(0)
    h = x_ref[...] @ wp_ref[...]
    z0 = h @ w0_ref[...]
    z1 = h @ w1_ref[...]
    zs_ref[0] = z0
    zs_ref[1] = z1
    el0 = z0 @ al0_ref[...].T
    er0 = z0 @ ar0_ref[...].T
    el1 = z1 @ al1_ref[...].T
    er1 = z1 @ ar1_ref[...].T
    cols = jnp.concatenate(
        [el0, er0, el1, er1, jnp.zeros((BLK, 12), jnp.float32)], axis=1)
    elr_ref[...] = cols
    m = jnp.max(cols, axis=0)[None, :]
    prev = jnp.where(i == 0, jnp.full((1, 16), -jnp.inf, jnp.float32),
                     mx_ref[...])
    mx = jnp.maximum(m, prev)
    mx_ref[...] = mx
    g0 = mx[0, 0] + mx[0, 1]
    g1 = mx[0, 2] + mx[0, 3]
    g0 = jnp.maximum(g0, 0.2 * g0)
    g1 = jnp.maximum(g1, 0.2 * g1)
    gmax_ref[...] = jnp.stack([jnp.full((16,), g0, jnp.float32),
                               jnp.full((16,), g1, jnp.float32)])


def _dense_pre(x, W_proj, W_fc0, W_fc1, al0, ar0, al1, ar1):
    return pl.pallas_call(
        _dense_pre_body,
        grid=(GRID,),
        in_specs=[
            pl.BlockSpec((BLK, D), lambda i: (i, 0)),
            pl.BlockSpec((D, D), lambda i: (0, 0)),
            pl.BlockSpec((D, D), lambda i: (0, 0)),
            pl.BlockSpec((D, D), lambda i: (0, 0)),
            pl.BlockSpec((1, D), lambda i: (0, 0)),
            pl.BlockSpec((1, D), lambda i: (0, 0)),
            pl.BlockSpec((1, D), lambda i: (0, 0)),
            pl.BlockSpec((1, D), lambda i: (0, 0)),
        ],
        out_specs=[
            pl.BlockSpec((2, BLK, D), lambda i: (0, i, 0)),
            pl.BlockSpec((BLK, 16), lambda i: (i, 0)),
            pl.BlockSpec((2, 16), lambda i: (0, 0)),
        ],
        out_shape=[
            jax.ShapeDtypeStruct((2, N, D), jnp.float32),
            jax.ShapeDtypeStruct((N, 16), jnp.float32),
            jax.ShapeDtypeStruct((2, 16), jnp.float32),
        ],
        scratch_shapes=[pltpu.VMEM((1, 16), jnp.float32)],
    )(x, W_proj, W_fc0, W_fc1, al0, ar0, al1, ar1)


def _sc_body(zs_h, ed_h, elr_h, gmax_h, zu_h, zs0_h,
             u_out, s_out,
             gmax_v, a_v,
             srcc0, srcc1, dstc0, dstc1, elb0, elb1, erb0, erb1,
             zb0, zb1, sr0, sr1,
             U_sh, S_sh,
             semi0, semi1, semz0, semz1, seml0, seml1, semr0, semr1,
             semU0, semU1, semS0, semS1):
    c = lax.axis_index("c")
    s = lax.axis_index("s")
    r0 = s * RPT
    B = ((srcc0, dstc0, elb0, erb0, zb0, sr0,
          semi0, semz0, seml0, semr0, semU0, semS0),
         (srcc1, dstc1, elb1, erb1, zb1, sr1,
          semi1, semz1, seml1, semr1, semU1, semS1))

    pltpu.sync_copy(gmax_h.at[c], gmax_v)
    pltpu.sync_copy(zu_h.at[pl.ds(r0, RPT)], U_sh.at[pl.ds(r0, RPT)])
    pltpu.sync_copy(zs0_h.at[pl.ds(r0, RPT)], S_sh.at[pl.ds(r0, RPT)])
    pltpu.sync_copy(zs0_h.at[pl.ds(0, CH)], sr0)
    pltpu.sync_copy(zs0_h.at[pl.ds(0, CH)], sr1)

    gm = gmax_v[...]
    iota16 = lax.iota(jnp.int32, 16)
    zi16 = jnp.zeros((16,), jnp.int32)
    cel = jnp.broadcast_to(2 * c, (16,)).astype(jnp.int32)
    cer = cel + 1

    plsc.subcore_barrier()

    def ckid(t):
        return s + t * NTILE

    def base_of(ck):
        return jnp.where(ck < NCK, ck, 0) * CH

    def issue_idx(t, p):
        (srcc, dstc, _, _, _, _, semi, *_), b = B[p], base_of(ckid(t))
        pltpu.async_copy(ed_h.at[c, 0, pl.ds(b, CH)], srcc, semi)
        pltpu.async_copy(ed_h.at[c, 1, pl.ds(b, CH)], dstc, semi)

    def wait_idx_issue_gathers(t, p):
        (srcc, dstc, elb, erb, zb, _, semi, semz, seml, semr, *_) = B[p]
        b = base_of(ckid(t))
        pltpu.make_async_copy(ed_h.at[c, 0, pl.ds(b, CH)], srcc, semi).wait()
        pltpu.make_async_copy(ed_h.at[c, 1, pl.ds(b, CH)], dstc, semi).wait()
        pltpu.async_copy(zs_h.at[c].at[srcc], zb, semz)
        pltpu.async_copy(elr_h.at[srcc], elb, seml)
        pltpu.async_copy(elr_h.at[dstc], erb, semr)

    def drain_scatters(p):
        (_, dstc, _, _, zb, sr, _, _, _, _, semU, semS) = B[p]
        pltpu.make_async_copy(zb, U_sh.at[dstc], semU).wait()
        pltpu.make_async_copy(sr, S_sh.at[dstc], semS).wait()

    def process(t, p, drain_pred, has_next):
        (srcc, dstc, elb, erb, zb, sr,
         semi, semz, seml, semr, semU, semS) = B[p]
        q = 1 - p
        valid = ckid(t) < NCK

        @pl.when(drain_pred)
        def _():
            drain_scatters(q)

        @pl.when(has_next)
        def _():
            issue_idx(t + 1, q)

        pltpu.make_async_copy(elr_h.at[srcc], elb, seml).wait()
        pltpu.make_async_copy(elr_h.at[dstc], erb, semr).wait()
        for j in range(CH // 16):
            ridx = iota16 + j * 16
            el = plsc.load_gather(elb, [ridx, cel])
            er = plsc.load_gather(erb, [ridx, cer])
            xx = el + er
            e = jnp.maximum(xx, 0.2 * xx)
            a = jnp.where(valid, jnp.exp(e - gm), 0.0)
            a_v[pl.ds(j * 16, 16)] = a
            plsc.store_scatter(sr, [ridx, zi16], a)

        @pl.when(has_next)
        def _():
            wait_idx_issue_gathers(t + 1, q)

        pltpu.make_async_copy(zs_h.at[c].at[srcc], zb, semz).wait()

        def scale_row(i, _):
            asp = plsc.load_gather(a_v, [jnp.broadcast_to(i, (16,))])
            for kk in range(D // 16):
                zb[i, pl.ds(kk * 16, 16)] = zb[i, pl.ds(kk * 16, 16)] * asp
            return 0
        lax.fori_loop(0, CH, scale_row, 0)

        pltpu.async_copy(zb, U_sh.at[dstc], semU, add=True)
        pltpu.async_copy(sr, S_sh.at[dstc], semS, add=True)

    issue_idx(0, 0)
    wait_idx_issue_gathers(0, 0)

    def pair(i, _):
        process(2 * i, 0, drain_pred=i >= 1, has_next=True)
        process(2 * i + 1, 1, drain_pred=True, has_next=True)
        return 0
    lax.fori_loop(0, (TPC - 1) // 2, pair, 0)
    process(TPC - 1, 0, drain_pred=True, has_next=False)
    drain_scatters(0)

    plsc.subcore_barrier()
    pltpu.sync_copy(U_sh.at[pl.ds(r0, RPT)], u_out.at[c, pl.ds(r0, RPT)])
    pltpu.sync_copy(S_sh.at[pl.ds(r0, RPT)], s_out.at[c, pl.ds(r0, RPT)])


def _sc_gat(zs, ed, elr, gmax, zu, zs0):
    mesh = plsc.VectorSubcoreMesh(core_axis_name="c", subcore_axis_name="s")
    dbl = lambda *sh: [sh, sh]
    f = pl.kernel(
        _sc_body,
        out_type=(jax.ShapeDtypeStruct((2, N, D), jnp.float32),
                  jax.ShapeDtypeStruct((2, N, SW), jnp.float32)),
        mesh=mesh,
        compiler_params=pltpu.CompilerParams(use_tc_tiling_on_sc=False,
                                             needs_layout_passes=False),
        scratch_types=(
            [pltpu.VMEM((16,), jnp.float32),
             pltpu.VMEM((CH,), jnp.float32)]
            + [pltpu.VMEM((CH,), jnp.int32)] * 4
            + [pltpu.VMEM((CH, 16), jnp.float32)] * 4
            + [pltpu.VMEM((CH, D), jnp.float32)] * 2
            + [pltpu.VMEM((CH, SW), jnp.float32)] * 2
            + [pltpu.VMEM_SHARED((N, D), jnp.float32),
               pltpu.VMEM_SHARED((N, SW), jnp.float32)]
            + [pltpu.SemaphoreType.DMA] * 12
        ),
    )
    return f(zs, ed, elr, gmax, zu, zs0)


def _z_from(u_ref, s_ref, mp):
    z = u_ref[mp] / (s_ref[mp, :, 0][:, None] + 1e-9)
    return jnp.where(z > 0, z, jnp.exp(jnp.minimum(z, 0.0)) - 1.0)


def _sem_w_body(u_ref, s_ref, ws1_ref, b_ref, ws2t_ref, w_ref, acc_ref):
    i = pl.program_id(0)
    z0 = _z_from(u_ref, s_ref, 0)
    z1 = _z_from(u_ref, s_ref, 1)
    t0 = jnp.sum(jnp.tanh(z0 @ ws1_ref[...] + b_ref[...]) * ws2t_ref[...])
    t1 = jnp.sum(jnp.tanh(z1 @ ws1_ref[...] + b_ref[...]) * ws2t_ref[...])
    cur = jnp.stack([t0, t1])[None, :]
    prev = jnp.where(i == 0, jnp.zeros((1, 2), jnp.float32), acc_ref[...])
    acc = prev + cur
    acc_ref[...] = acc
    w_ref[...] = acc / N


def _combine_body(u_ref, s_ref, w_ref, out_ref):
    z0 = _z_from(u_ref, s_ref, 0)
    z1 = _z_from(u_ref, s_ref, 1)
    w0 = w_ref[0, 0]
    w1 = w_ref[0, 1]
    m = jnp.maximum(w0, w1)
    b0 = jnp.exp(w0 - m)
    b1 = jnp.exp(w1 - m)
    t = b0 + b1
    out_ref[...] = (b0 / t) * z0 + (b1 / t) * z1


def _combine(u, s_, W_s1, b_s1, W_s2):
    b2d = b_s1.reshape(1, D)
    ws2t = W_s2.reshape(1, D)
    u_spec = pl.BlockSpec((2, BLK, D), lambda i: (0, i, 0))
    s_spec = pl.BlockSpec((2, BLK, SW), lambda i: (0, i, 0))
    w = pl.pallas_call(
        _sem_w_body,
        grid=(GRID,),
        in_specs=[
            u_spec, s_spec,
            pl.BlockSpec((D, D), lambda i: (0, 0)),
            pl.BlockSpec((1, D), lambda i: (0, 0)),
            pl.BlockSpec((1, D), lambda i: (0, 0)),
        ],
        out_specs=pl.BlockSpec((1, 2), lambda i: (0, 0)),
        out_shape=jax.ShapeDtypeStruct((1, 2), jnp.float32),
        scratch_shapes=[pltpu.VMEM((1, 2), jnp.float32)],
    )(u, s_, W_s1, b2d, ws2t)
    return pl.pallas_call(
        _combine_body,
        grid=(GRID,),
        in_specs=[u_spec, s_spec, pl.BlockSpec((1, 2), lambda i: (0, 0))],
        out_specs=pl.BlockSpec((BLK, D), lambda i: (i, 0)),
        out_shape=jax.ShapeDtypeStruct((N, D), jnp.float32),
    )(u, s_, w)


def kernel(x, edge_index_mp0, edge_index_mp1, W_proj, W_fc0, attn_l0,
           attn_r0, W_fc1, attn_l1, attn_r1, W_s1, b_s1, W_s2):
    zs, elr, gmax = _dense_pre(x, W_proj, W_fc0, W_fc1,
                               attn_l0, attn_r0, attn_l1, attn_r1)
    zu = jnp.zeros((N, D), jnp.float32)
    zs0 = jnp.zeros((N, SW), jnp.float32)
    ed = jnp.stack([edge_index_mp0, edge_index_mp1])
    u, s_ = _sc_gat(zs, ed, elr, gmax, zu, zs0)
    return _combine(u, s_, W_s1, b_s1, W_s2)


# speedup vs baseline: 66.3294x; 1.2492x over previous
import functools

import jax
import jax.numpy as jnp
from jax import lax
from jax.experimental import pallas as pl
from jax.experimental.pallas import tpu as pltpu
from jax.experimental.pallas import tpu_sc as plsc

N = 10000
E = 320000
D = 128
BLK = 1000
GRID = N // BLK

NTILE = 16
CH = 128
NCK = E // CH
TPC = -(-NCK // NTILE)
SW = 8
RPT = N // NTILE


def _dense_pre_body(x_ref, wp_ref, w0_ref, w1_ref, al0_ref, ar0_ref,
                    al1_ref, ar1_ref, zs_ref, elr_ref, gmax_ref, mx_ref):
    i = pl.program_id(0)
    h = x_ref[...] @ wp_ref[...]
    z0 = h @ w0_ref[...]
    z1 = h @ w1_ref[...]
    zs_ref[0] = z0
    zs_ref[1] = z1
    el0 = z0 @ al0_ref[...].T
    er0 = z0 @ ar0_ref[...].T
    el1 = z1 @ al1_ref[...].T
    er1 = z1 @ ar1_ref[...].T
    cols = jnp.concatenate(
        [el0, er0, el1, er1, jnp.zeros((BLK, 12), jnp.float32)], axis=1)
    elr_ref[...] = cols
    m = jnp.max(cols, axis=0)[None, :]
    prev = jnp.where(i == 0, jnp.full((1, 16), -jnp.inf, jnp.float32),
                     mx_ref[...])
    mx = jnp.maximum(m, prev)
    mx_ref[...] = mx
    g0 = mx[0, 0] + mx[0, 1]
    g1 = mx[0, 2] + mx[0, 3]
    g0 = jnp.maximum(g0, 0.2 * g0)
    g1 = jnp.maximum(g1, 0.2 * g1)
    gmax_ref[...] = jnp.stack([jnp.full((16,), g0, jnp.float32),
                               jnp.full((16,), g1, jnp.float32)])


def _dense_pre(x, W_proj, W_fc0, W_fc1, al0, ar0, al1, ar1):
    return pl.pallas_call(
        _dense_pre_body,
        grid=(GRID,),
        in_specs=[
            pl.BlockSpec((BLK, D), lambda i: (i, 0)),
            pl.BlockSpec((D, D), lambda i: (0, 0)),
            pl.BlockSpec((D, D), lambda i: (0, 0)),
            pl.BlockSpec((D, D), lambda i: (0, 0)),
            pl.BlockSpec((1, D), lambda i: (0, 0)),
            pl.BlockSpec((1, D), lambda i: (0, 0)),
            pl.BlockSpec((1, D), lambda i: (0, 0)),
            pl.BlockSpec((1, D), lambda i: (0, 0)),
        ],
        out_specs=[
            pl.BlockSpec((2, BLK, D), lambda i: (0, i, 0)),
            pl.BlockSpec((BLK, 16), lambda i: (i, 0)),
            pl.BlockSpec((2, 16), lambda i: (0, 0)),
        ],
        out_shape=[
            jax.ShapeDtypeStruct((2, N, D), jnp.float32),
            jax.ShapeDtypeStruct((N, 16), jnp.float32),
            jax.ShapeDtypeStruct((2, 16), jnp.float32),
        ],
        scratch_shapes=[pltpu.VMEM((1, 16), jnp.float32)],
    )(x, W_proj, W_fc0, W_fc1, al0, ar0, al1, ar1)


def _sc_body(zs_h, ed_h, elr_h, gmax_h, zu_h, zs0_h,
             u_out, s_out,
             gmax_v, a_v,
             srcc0, srcc1, dstc0, dstc1, elb0, elb1, erb0, erb1,
             zb0, zb1, sr0, sr1,
             U_sh, S_sh,
             semi0, semi1, semz0, semz1, seml0, seml1, semr0, semr1,
             semU0, semU1, semS0, semS1):
    c = lax.axis_index("c")
    s = lax.axis_index("s")
    r0 = s * RPT
    B = ((srcc0, dstc0, elb0, erb0, zb0, sr0,
          semi0, semz0, seml0, semr0, semU0, semS0),
         (srcc1, dstc1, elb1, erb1, zb1, sr1,
          semi1, semz1, seml1, semr1, semU1, semS1))

    pltpu.sync_copy(gmax_h.at[c], gmax_v)
    pltpu.sync_copy(zu_h.at[pl.ds(r0, RPT)], U_sh.at[pl.ds(r0, RPT)])
    pltpu.sync_copy(zs0_h.at[pl.ds(r0, RPT)], S_sh.at[pl.ds(r0, RPT)])
    pltpu.sync_copy(zs0_h.at[pl.ds(0, CH)], sr0)
    pltpu.sync_copy(zs0_h.at[pl.ds(0, CH)], sr1)

    gm = gmax_v[...]
    iota16 = lax.iota(jnp.int32, 16)
    zi16 = jnp.zeros((16,), jnp.int32)
    cel = jnp.broadcast_to(2 * c, (16,)).astype(jnp.int32)
    cer = cel + 1

    plsc.subcore_barrier()

    def ckid(t):
        return s + t * NTILE

    def base_of(ck):
        return jnp.where(ck < NCK, ck, 0) * CH

    def issue_idx(t, p):
        (srcc, dstc, _, _, _, _, semi, *_), b = B[p], base_of(ckid(t))
        pltpu.async_copy(ed_h.at[c, 0, pl.ds(b, CH)], srcc, semi)
        pltpu.async_copy(ed_h.at[c, 1, pl.ds(b, CH)], dstc, semi)

    def wait_idx_issue_gathers(t, p):
        (srcc, dstc, elb, erb, zb, _, semi, semz, seml, semr, *_) = B[p]
        b = base_of(ckid(t))
        pltpu.make_async_copy(ed_h.at[c, 0, pl.ds(b, CH)], srcc, semi).wait()
        pltpu.make_async_copy(ed_h.at[c, 1, pl.ds(b, CH)], dstc, semi).wait()
        pltpu.async_copy(zs_h.at[c].at[srcc], zb, semz)
        pltpu.async_copy(elr_h.at[srcc], elb, seml)
        pltpu.async_copy(elr_h.at[dstc], erb, semr)

    def drain_scatters(p):
        (_, dstc, _, _, zb, sr, _, _, _, _, semU, semS) = B[p]
        pltpu.make_async_copy(zb, U_sh.at[dstc], semU).wait()
        pltpu.make_async_copy(sr, S_sh.at[dstc], semS).wait()

    def process(t, p, drain_pred, has_next):
        (srcc, dstc, elb, erb, zb, sr,
         semi, semz, seml, semr, semU, semS) = B[p]
        q = 1 - p
        valid = ckid(t) < NCK

        @pl.when(drain_pred)
        def _():
            drain_scatters(q)

        @pl.when(has_next)
        def _():
            issue_idx(t + 1, q)

        pltpu.make_async_copy(elr_h.at[srcc], elb, seml).wait()
        pltpu.make_async_copy(elr_h.at[dstc], erb, semr).wait()
        for j in range(CH // 16):
            ridx = iota16 + j * 16
            el = plsc.load_gather(elb, [ridx, cel])
            er = plsc.load_gather(erb, [ridx, cer])
            xx = el + er
            e = jnp.maximum(xx, 0.2 * xx)
            a = jnp.where(valid, jnp.exp(e - gm), 0.0)
            a_v[pl.ds(j * 16, 16)] = a
            plsc.store_scatter(sr, [ridx, zi16], a)

        @pl.when(has_next)
        def _():
            wait_idx_issue_gathers(t + 1, q)

        pltpu.make_async_copy(zs_h.at[c].at[srcc], zb, semz).wait()

        @plsc.parallel_loop(0, CH, 1, unroll=4)
        def _(i):
            asp = plsc.load_gather(a_v, [jnp.broadcast_to(i, (16,))])
            for kk in range(D // 16):
                zb[i, pl.ds(kk * 16, 16)] = zb[i, pl.ds(kk * 16, 16)] * asp

        pltpu.async_copy(zb, U_sh.at[dstc], semU, add=True)
        pltpu.async_copy(sr, S_sh.at[dstc], semS, add=True)

    issue_idx(0, 0)
    wait_idx_issue_gathers(0, 0)

    def pair(i, _):
        process(2 * i, 0, drain_pred=i >= 1, has_next=True)
        process(2 * i + 1, 1, drain_pred=True, has_next=True)
        return 0
    lax.fori_loop(0, (TPC - 1) // 2, pair, 0)
    process(TPC - 1, 0, drain_pred=True, has_next=False)
    drain_scatters(0)

    plsc.subcore_barrier()
    pltpu.sync_copy(U_sh.at[pl.ds(r0, RPT)], u_out.at[c, pl.ds(r0, RPT)])
    pltpu.sync_copy(S_sh.at[pl.ds(r0, RPT)], s_out.at[c, pl.ds(r0, RPT)])


def _sc_gat(zs, ed, elr, gmax, zu, zs0):
    mesh = plsc.VectorSubcoreMesh(core_axis_name="c", subcore_axis_name="s")
    dbl = lambda *sh: [sh, sh]
    f = pl.kernel(
        _sc_body,
        out_type=(jax.ShapeDtypeStruct((2, N, D), jnp.float32),
                  jax.ShapeDtypeStruct((2, N, SW), jnp.float32)),
        mesh=mesh,
        compiler_params=pltpu.CompilerParams(use_tc_tiling_on_sc=False,
                                             needs_layout_passes=False),
        scratch_types=(
            [pltpu.VMEM((16,), jnp.float32),
             pltpu.VMEM((CH,), jnp.float32)]
            + [pltpu.VMEM((CH,), jnp.int32)] * 4
            + [pltpu.VMEM((CH, 16), jnp.float32)] * 4
            + [pltpu.VMEM((CH, D), jnp.float32)] * 2
            + [pltpu.VMEM((CH, SW), jnp.float32)] * 2
            + [pltpu.VMEM_SHARED((N, D), jnp.float32),
               pltpu.VMEM_SHARED((N, SW), jnp.float32)]
            + [pltpu.SemaphoreType.DMA] * 12
        ),
    )
    return f(zs, ed, elr, gmax, zu, zs0)


def _z_from(u_ref, s_ref, mp):
    z = u_ref[mp] / (s_ref[mp, :, 0][:, None] + 1e-9)
    return jnp.where(z > 0, z, jnp.exp(jnp.minimum(z, 0.0)) - 1.0)


def _sem_w_body(u_ref, s_ref, ws1_ref, b_ref, ws2t_ref, w_ref, acc_ref):
    i = pl.program_id(0)
    z0 = _z_from(u_ref, s_ref, 0)
    z1 = _z_from(u_ref, s_ref, 1)
    t0 = jnp.sum(jnp.tanh(z0 @ ws1_ref[...] + b_ref[...]) * ws2t_ref[...])
    t1 = jnp.sum(jnp.tanh(z1 @ ws1_ref[...] + b_ref[...]) * ws2t_ref[...])
    cur = jnp.stack([t0, t1])[None, :]
    prev = jnp.where(i == 0, jnp.zeros((1, 2), jnp.float32), acc_ref[...])
    acc = prev + cur
    acc_ref[...] = acc
    w_ref[...] = acc / N


def _combine_body(u_ref, s_ref, w_ref, out_ref):
    z0 = _z_from(u_ref, s_ref, 0)
    z1 = _z_from(u_ref, s_ref, 1)
    w0 = w_ref[0, 0]
    w1 = w_ref[0, 1]
    m = jnp.maximum(w0, w1)
    b0 = jnp.exp(w0 - m)
    b1 = jnp.exp(w1 - m)
    t = b0 + b1
    out_ref[...] = (b0 / t) * z0 + (b1 / t) * z1


def _combine(u, s_, W_s1, b_s1, W_s2):
    b2d = b_s1.reshape(1, D)
    ws2t = W_s2.reshape(1, D)
    u_spec = pl.BlockSpec((2, BLK, D), lambda i: (0, i, 0))
    s_spec = pl.BlockSpec((2, BLK, SW), lambda i: (0, i, 0))
    w = pl.pallas_call(
        _sem_w_body,
        grid=(GRID,),
        in_specs=[
            u_spec, s_spec,
            pl.BlockSpec((D, D), lambda i: (0, 0)),
            pl.BlockSpec((1, D), lambda i: (0, 0)),
            pl.BlockSpec((1, D), lambda i: (0, 0)),
        ],
        out_specs=pl.BlockSpec((1, 2), lambda i: (0, 0)),
        out_shape=jax.ShapeDtypeStruct((1, 2), jnp.float32),
        scratch_shapes=[pltpu.VMEM((1, 2), jnp.float32)],
    )(u, s_, W_s1, b2d, ws2t)
    return pl.pallas_call(
        _combine_body,
        grid=(GRID,),
        in_specs=[u_spec, s_spec, pl.BlockSpec((1, 2), lambda i: (0, 0))],
        out_specs=pl.BlockSpec((BLK, D), lambda i: (i, 0)),
        out_shape=jax.ShapeDtypeStruct((N, D), jnp.float32),
    )(u, s_, w)


def kernel(x, edge_index_mp0, edge_index_mp1, W_proj, W_fc0, attn_l0,
           attn_r0, W_fc1, attn_l1, attn_r1, W_s1, b_s1, W_s2):
    zs, elr, gmax = _dense_pre(x, W_proj, W_fc0, W_fc1,
                               attn_l0, attn_r0, attn_l1, attn_r1)
    zu = jnp.zeros((N, D), jnp.float32)
    zs0 = jnp.zeros((N, SW), jnp.float32)
    ed = jnp.stack([edge_index_mp0, edge_index_mp1])
    u, s_ = _sc_gat(zs, ed, elr, gmax, zu, zs0)
    return _combine(u, s_, W_s1, b_s1, W_s2)
